# baseline (speedup 1.0000x reference)
import functools

import jax
import jax.numpy as jnp
from jax import lax
from jax.experimental import pallas as pl
from jax.experimental.pallas import tpu as pltpu
from jax.experimental.pallas import tpu_sc as plsc

NC = 2
NS = 16
NW = NC * NS
L = 16

CHUNK = 512


def _make_gather(B, D):
    n_per_w = B // NW
    n_chunks = n_per_w // CHUNK
    mesh = plsc.VectorSubcoreMesh(core_axis_name="c", subcore_axis_name="s")

    @functools.partial(
        pl.kernel,
        mesh=mesh,
        out_type=jax.ShapeDtypeStruct((B, D), jnp.float32),
        scratch_types=[
            pltpu.VMEM((CHUNK,), jnp.int32),
            pltpu.SemaphoreType.DMA,
            pltpu.SemaphoreType.DMA,
        ],
    )
    def k(x_hbm, idx_hbm, out_hbm, idx_v, isem, rsem):
        wid = lax.axis_index("s") * NC + lax.axis_index("c")
        row0 = wid * n_per_w

        def chunk_body(c, carry):
            base = row0 + c * CHUNK
            pltpu.async_copy(
                idx_hbm.at[pl.ds(base, CHUNK)], idx_v, isem).wait()

            def group_body(g, carry2):
                vec = idx_v[pl.ds(g * L, L)]

                def issue(j, i):
                    pltpu.async_copy(
                        x_hbm.at[i], out_hbm.at[base + g * L + j], rsem)

                for j in range(L):
                    issue(j, lax.squeeze(lax.slice(vec, (j,), (j + 1,)),
                                         dimensions=(0,)))
                return carry2

            lax.fori_loop(0, CHUNK // L, group_body, 0)
            return carry

        lax.fori_loop(0, n_chunks, chunk_body, 0)
        pltpu.make_async_copy(
            x_hbm.at[pl.ds(0, n_per_w)],
            out_hbm.at[pl.ds(row0, n_per_w)], rsem).wait()

    return k


def kernel(x, dim, idx):
    B = idx.shape[0]
    D = x.shape[1]
    idx32 = (idx + dim).astype(jnp.int32)
    return _make_gather(B, D)(x, idx32)


# speedup vs baseline: 7.3299x; 7.3299x over previous
import functools

import jax
import jax.numpy as jnp
from jax import lax
from jax.experimental import pallas as pl
from jax.experimental.pallas import tpu as pltpu
from jax.experimental.pallas import tpu_sc as plsc

NC = 2
NS = 16
NW = NC * NS
L = 16

CHUNK = 128


def _make_gather(B, D):
    n_per_w = B // NW
    n_chunks = n_per_w // CHUNK
    mesh = plsc.VectorSubcoreMesh(core_axis_name="c", subcore_axis_name="s")

    @functools.partial(
        pl.kernel,
        mesh=mesh,
        out_type=jax.ShapeDtypeStruct((B, D), jnp.float32),
        compiler_params=pltpu.CompilerParams(needs_layout_passes=False),
        scratch_types=[
            pltpu.VMEM((n_per_w,), jnp.int32),
            pltpu.VMEM((2, CHUNK), jnp.int32),
            pltpu.VMEM((2, CHUNK, 2 * D), jnp.float32),
            pltpu.VMEM((2, CHUNK, D), jnp.float32),
            pltpu.SemaphoreType.DMA,
            pltpu.SemaphoreType.DMA,
            pltpu.SemaphoreType.DMA,
            pltpu.SemaphoreType.DMA,
        ],
    )
    def k(xp_hbm, idx_hbm, out_hbm, idx_v, pidx_v, pairs_v, rows_v, *sems):
        g0, g1, s0, s1 = sems
        gsems = (g0, g1)
        ssems = (s0, s1)
        wid = lax.axis_index("s") * NC + lax.axis_index("c")
        row0 = wid * n_per_w
        pltpu.sync_copy(idx_hbm.at[pl.ds(row0, n_per_w)], idx_v)

        def fire(c, b):
            for g in range(CHUNK // L):
                vec = idx_v[pl.ds(c * CHUNK + g * L, L)]
                pidx_v[b, pl.ds(g * L, L)] = jax.lax.shift_right_logical(vec, 1)
            pltpu.async_copy(xp_hbm.at[pidx_v.at[b]], pairs_v.at[b], gsems[b])

        def drain_gather(b):
            pltpu.make_async_copy(
                xp_hbm.at[pl.ds(0, CHUNK)], pairs_v.at[b], gsems[b]).wait()

        def select_store(c, b):
            def group(g, carry):
                vec = idx_v[pl.ds(c * CHUNK + g * L, L)]
                offs = jax.lax.bitwise_and(vec, 1) * D
                for j in range(L):
                    o = lax.squeeze(lax.slice(offs, (j,), (j + 1,)),
                                    dimensions=(0,))
                    r = g * L + j
                    for jb in range(D // L):
                        col = o + jb * L
                        v = plsc.load_gather(
                            pairs_v.at[b],
                            [jnp.full((L,), r, jnp.int32),
                             col + lax.iota(jnp.int32, L)])
                        rows_v[b, r, pl.ds(jb * L, L)] = v
                return carry

            lax.fori_loop(0, CHUNK // L, group, 0)
            pltpu.async_copy(
                rows_v.at[b],
                out_hbm.at[pl.ds(row0 + c * CHUNK, CHUNK)], ssems[b])

        def drain_store(b):
            pltpu.make_async_copy(
                rows_v.at[b], out_hbm.at[pl.ds(0, CHUNK)], ssems[b]).wait()

        fire(0, 0)

        def body(i, carry):
            c = i * 2
            for b in range(2):

                @pl.when(c + b + 1 < n_chunks)
                def _():
                    fire(c + b + 1, 1 - b)

                drain_gather(b)

                @pl.when(c + b >= 2)
                def _():
                    drain_store(b)

                select_store(c + b, b)
            return carry

        lax.fori_loop(0, n_chunks // 2, body, 0)
        drain_store(0)
        drain_store(1)

    return k


def kernel(x, dim, idx):
    B = idx.shape[0]
    D = x.shape[1]
    idx32 = (idx + dim).astype(jnp.int32)
    xp = x.reshape(x.shape[0] // 2, 2 * D)
    return _make_gather(B, D)(xp, idx32)
